# Initial kernel scaffold; baseline (speedup 1.0000x reference)
import jax
import jax.numpy as jnp
from jax.experimental import pallas as pl


def kernel(x, batch, v, lin):
    raise NotImplementedError("write your pallas kernel here")


# speedup vs baseline: 61.6006x; 61.6006x over previous
import jax
import jax.numpy as jnp
from jax.experimental import pallas as pl

N = 50000
F = 128
T = 32
S = 32
B = 128

TN = 2000
GRID = N // TN


def _ect_kernel(x_ref, b_ref, v_ref, lin_ref, o_ref):
    i = pl.program_id(0)
    nh = jnp.dot(x_ref[...], v_ref[...], preferred_element_type=jnp.float32)
    nh_t = jnp.tile(nh, (1, S))
    ecc = jax.nn.sigmoid(lin_ref[0:1, :] - 200.0 * nh_t)
    bid = b_ref[0]
    iota_b = jax.lax.broadcasted_iota(jnp.int32, (B, TN), 0)
    onehot = (iota_b == bid).astype(jnp.float32)
    part = jnp.dot(onehot, ecc, preferred_element_type=jnp.float32)

    @pl.when(i == 0)
    def _init():
        o_ref[...] = part

    @pl.when(i > 0)
    def _acc():
        o_ref[...] += part


def kernel(x, batch, v, lin):
    batch3d = batch.reshape(GRID, 1, TN)
    lin200 = jnp.broadcast_to(
        (200.0 * jnp.repeat(lin.reshape(-1), T)).reshape(1, S * T), (8, S * T)
    )
    out2d = pl.pallas_call(
        _ect_kernel,
        grid=(GRID,),
        in_specs=[
            pl.BlockSpec((TN, F), lambda i: (i, 0)),
            pl.BlockSpec((1, 1, TN), lambda i: (i, 0, 0)),
            pl.BlockSpec((F, T), lambda i: (0, 0)),
            pl.BlockSpec((8, S * T), lambda i: (0, 0)),
        ],
        out_specs=pl.BlockSpec((B, S * T), lambda i: (0, 0)),
        out_shape=jax.ShapeDtypeStruct((B, S * T), jnp.float32),
    )(x, batch3d, v, lin200)
    return out2d.reshape(B, S, T)


# speedup vs baseline: 62.1821x; 1.0094x over previous
import jax
import jax.numpy as jnp
from jax.experimental import pallas as pl

N = 50000
F = 128
T = 32
S = 32
B = 128

TN = 2000
GRID = N // TN


def _ect_kernel(x_ref, b_ref, v_ref, lin_ref, o_ref):
    i = pl.program_id(0)
    nh = jnp.dot(x_ref[...], v_ref[...], preferred_element_type=jnp.float32)
    nh_t = jnp.tile(nh, (1, S))
    z = (lin_ref[0:1, :] - 200.0 * nh_t).astype(jnp.bfloat16)
    ecc = jax.nn.sigmoid(z)
    bid = b_ref[0]
    iota_b = jax.lax.broadcasted_iota(jnp.int32, (B, TN), 0)
    onehot = (iota_b == bid).astype(jnp.bfloat16)
    part = jnp.dot(onehot, ecc, preferred_element_type=jnp.float32)

    @pl.when(i == 0)
    def _init():
        o_ref[...] = part

    @pl.when(i > 0)
    def _acc():
        o_ref[...] += part


def kernel(x, batch, v, lin):
    batch3d = batch.reshape(GRID, 1, TN)
    lin200 = jnp.broadcast_to(
        (200.0 * jnp.repeat(lin.reshape(-1), T)).reshape(1, S * T), (8, S * T)
    )
    out2d = pl.pallas_call(
        _ect_kernel,
        grid=(GRID,),
        in_specs=[
            pl.BlockSpec((TN, F), lambda i: (i, 0)),
            pl.BlockSpec((1, 1, TN), lambda i: (i, 0, 0)),
            pl.BlockSpec((F, T), lambda i: (0, 0)),
            pl.BlockSpec((8, S * T), lambda i: (0, 0)),
        ],
        out_specs=pl.BlockSpec((B, S * T), lambda i: (0, 0)),
        out_shape=jax.ShapeDtypeStruct((B, S * T), jnp.float32),
    )(x, batch3d, v, lin200)
    return out2d.reshape(B, S, T)


# speedup vs baseline: 105.6040x; 1.6983x over previous
import jax
import jax.numpy as jnp
from jax.experimental import pallas as pl

N = 50000
F = 128
T = 32
S = 32
B = 128

TN = 5000
GRID = N // TN


def _ect_kernel(x_ref, b_ref, v_ref, lin_ref, o_ref):
    i = pl.program_id(0)
    nh = jnp.dot(x_ref[...], v_ref[...], preferred_element_type=jnp.float32)
    nh16 = nh.astype(jnp.bfloat16)
    w = lin_ref[0:1, :] + jnp.tile(nh16, (1, S))
    ecc = 1.0 + jnp.tanh(w)
    bid = b_ref[0]
    iota_b = jax.lax.broadcasted_iota(jnp.int32, (B, TN), 0)
    onehot = jnp.where(iota_b == bid, 0.5, 0.0).astype(jnp.bfloat16)
    part = jnp.dot(onehot, ecc, preferred_element_type=jnp.float32)

    @pl.when(i == 0)
    def _init():
        o_ref[...] = part

    @pl.when(i > 0)
    def _acc():
        o_ref[...] += part


def kernel(x, batch, v, lin):
    batch3d = batch.reshape(GRID, 1, TN)
    v100 = v * (-100.0)
    lin100 = jnp.broadcast_to(
        (100.0 * jnp.repeat(lin.reshape(-1), T)).reshape(1, S * T), (8, S * T)
    ).astype(jnp.bfloat16)
    out2d = pl.pallas_call(
        _ect_kernel,
        grid=(GRID,),
        in_specs=[
            pl.BlockSpec((TN, F), lambda i: (i, 0)),
            pl.BlockSpec((1, 1, TN), lambda i: (i, 0, 0)),
            pl.BlockSpec((F, T), lambda i: (0, 0)),
            pl.BlockSpec((8, S * T), lambda i: (0, 0)),
        ],
        out_specs=pl.BlockSpec((B, S * T), lambda i: (0, 0)),
        out_shape=jax.ShapeDtypeStruct((B, S * T), jnp.float32),
    )(x, batch3d, v100, lin100)
    return out2d.reshape(B, S, T)
